# Initial kernel scaffold; baseline (speedup 1.0000x reference)
import jax
import jax.numpy as jnp
from jax.experimental import pallas as pl


def kernel(token_ids, weight):
    raise NotImplementedError("write your pallas kernel here")


# speedup vs baseline: 1.1017x; 1.1017x over previous
import functools

import jax
import jax.numpy as jnp
from jax import lax
from jax.experimental import pallas as pl
from jax.experimental.pallas import tpu as pltpu
from jax.experimental.pallas import tpu_sc as plsc

_L = 128


def _make_gather(B, D, NW, G):
    C = G * _L
    b_per_w = B // NW
    n_chunks = b_per_w // C
    mesh = plsc.VectorSubcoreMesh(core_axis_name="c", subcore_axis_name="s")

    @functools.partial(
        pl.kernel,
        mesh=mesh,
        out_type=jax.ShapeDtypeStruct((B, D), jnp.float32),
        scratch_types=[
            pltpu.VMEM((G, _L), jnp.int32),
            pltpu.VMEM((C, D), jnp.float32),
            pltpu.SemaphoreType.DMA,
        ],
        compiler_params=pltpu.CompilerParams(use_tc_tiling_on_sc=False),
    )
    def gather_kernel(idx_hbm, table_hbm, out_hbm, idx_v, rows_v, sem):
        wid = lax.axis_index("s") * 2 + lax.axis_index("c")
        base = wid * b_per_w

        def body(g, carry):
            off = base + g * C
            pltpu.sync_copy(idx_hbm.at[pl.ds(off // _L, G)], idx_v)
            copies = []
            for j in range(G):
                copies.append(
                    pltpu.async_copy(
                        table_hbm.at[idx_v.at[j]],
                        rows_v.at[pl.ds(j * _L, _L)],
                        sem,
                    )
                )
            for c in copies:
                c.wait()
            pltpu.sync_copy(rows_v, out_hbm.at[pl.ds(off, C)])
            return carry

        lax.fori_loop(0, n_chunks, body, 0)

    return gather_kernel


def kernel(token_ids, weight):
    S0, S1 = token_ids.shape
    V, D = weight.shape
    B = S0 * S1
    NW = 32
    G = 8
    assert B % (NW * G * _L) == 0
    idx_2d = token_ids.reshape(B // _L, _L)
    out = _make_gather(B, D, NW, G)(idx_2d, weight)
    return out.reshape(S0, S1, D)


# speedup vs baseline: 1.1124x; 1.0097x over previous
import functools

import jax
import jax.numpy as jnp
from jax import lax
from jax.experimental import pallas as pl
from jax.experimental.pallas import tpu as pltpu
from jax.experimental.pallas import tpu_sc as plsc

_L = 128


def _make_gather(B, D, NW, G):
    C = G * _L
    b_per_w = B // NW
    n_chunks = b_per_w // C
    assert n_chunks % 2 == 0 and n_chunks >= 4
    n_pairs = n_chunks // 2
    mesh = plsc.VectorSubcoreMesh(core_axis_name="c", subcore_axis_name="s")

    @functools.partial(
        pl.kernel,
        mesh=mesh,
        out_type=jax.ShapeDtypeStruct((B, D), jnp.float32),
        scratch_types=[
            pltpu.VMEM((2, G, _L), jnp.int32),
            pltpu.VMEM((2, C, D), jnp.float32),
            pltpu.SemaphoreType.DMA,
            pltpu.SemaphoreType.DMA,
            pltpu.SemaphoreType.DMA,
            pltpu.SemaphoreType.DMA,
        ],
        compiler_params=pltpu.CompilerParams(use_tc_tiling_on_sc=False),
    )
    def gather_kernel(idx_hbm, table_hbm, out_hbm, idx_v, rows_v,
                      sg0, sg1, so0, so1):
        sg = (sg0, sg1)
        so = (so0, so1)
        wid = lax.axis_index("s") * 2 + lax.axis_index("c")
        base = wid * b_per_w

        def stage_idx(g, b):
            pltpu.sync_copy(
                idx_hbm.at[pl.ds((base + g * C) // _L, G)], idx_v.at[b])

        def fire_gathers(b):
            for j in range(G):
                pltpu.make_async_copy(
                    table_hbm.at[idx_v.at[b, j]],
                    rows_v.at[b, pl.ds(j * _L, _L)],
                    sg[b],
                ).start()

        def wait_gathers(b):
            for j in range(G):
                pltpu.make_async_copy(
                    table_hbm.at[idx_v.at[b, j]],
                    rows_v.at[b, pl.ds(j * _L, _L)],
                    sg[b],
                ).wait()

        def fire_out(g, b):
            pltpu.make_async_copy(
                rows_v.at[b], out_hbm.at[pl.ds(base + g * C, C)], so[b]
            ).start()

        def wait_out(g, b):
            pltpu.make_async_copy(
                rows_v.at[b], out_hbm.at[pl.ds(base + g * C, C)], so[b]
            ).wait()

        for b in (0, 1):
            stage_idx(b, b)
            fire_gathers(b)

        def pair(i, carry):
            for b in (0, 1):
                g = 2 * i + b
                wait_gathers(b)
                fire_out(g, b)
                stage_idx(g + 2, b)
                wait_out(g, b)
                fire_gathers(b)
            return carry

        lax.fori_loop(0, n_pairs - 1, pair, 0)

        for b in (0, 1):
            g = n_chunks - 2 + b
            wait_gathers(b)
            fire_out(g, b)
        for b in (0, 1):
            wait_out(n_chunks - 2 + b, b)

    return gather_kernel


def kernel(token_ids, weight):
    S0, S1 = token_ids.shape
    V, D = weight.shape
    B = S0 * S1
    NW = 32
    G = 10
    assert B % (NW * G * _L) == 0
    idx_2d = token_ids.reshape(B // _L, _L)
    out = _make_gather(B, D, NW, G)(idx_2d, weight)
    return out.reshape(S0, S1, D)


# speedup vs baseline: 3.1068x; 2.7929x over previous
import functools

import jax
import jax.numpy as jnp
from jax import lax
from jax.experimental import pallas as pl
from jax.experimental.pallas import tpu as pltpu
from jax.experimental.pallas import tpu_sc as plsc

_L = 128


def _make_gather_t(S1, S0, D, NW, R):
    G = R // _L
    n_ch = S0 // R
    n_tasks = S1 * n_ch
    per_w = n_tasks // NW
    assert n_tasks % NW == 0
    mesh = plsc.VectorSubcoreMesh(core_axis_name="c", subcore_axis_name="s")

    @functools.partial(
        pl.kernel,
        mesh=mesh,
        out_type=jax.ShapeDtypeStruct((S1, D, S0), jnp.float32),
        scratch_types=[
            pltpu.VMEM((G, _L), jnp.int32),
            pltpu.VMEM((R, D), jnp.float32),
            pltpu.VMEM((D, R), jnp.float32),
            pltpu.SemaphoreType.DMA,
        ],
        compiler_params=pltpu.CompilerParams(
            use_tc_tiling_on_sc=False, needs_layout_passes=False),
    )
    def gather_kernel(t2_hbm, table_hbm, out_hbm, idx_v, rows_v, tr_v, sem):
        wid = lax.axis_index("s") * 2 + lax.axis_index("c")
        iota16 = lax.iota(jnp.int32, 16)

        def task_body(i, carry):
            task = wid * per_w + i
            s1 = task // n_ch
            ch = task % n_ch
            pltpu.sync_copy(t2_hbm.at[s1, pl.ds(ch * G, G)], idx_v)
            for j in range(G):
                pltpu.make_async_copy(
                    table_hbm.at[idx_v.at[j]],
                    rows_v.at[pl.ds(j * _L, _L)],
                    sem,
                ).start()
            for j in range(G):
                pltpu.make_async_copy(
                    table_hbm.at[idx_v.at[j]],
                    rows_v.at[pl.ds(j * _L, _L)],
                    sem,
                ).wait()

            def c_body(c, _):
                col = jnp.full((16,), c, dtype=jnp.int32)

                def t_body(tb, __):
                    for u in range(4):
                        t0 = tb * 64 + u * 16
                        row = t0 + iota16
                        v = plsc.load_gather(rows_v, [row, col])
                        tr_v[c, pl.ds(t0, 16)] = v
                    return __

                lax.fori_loop(0, R // 64, t_body, 0)
                return _

            lax.fori_loop(0, D, c_body, 0)
            pltpu.sync_copy(tr_v, out_hbm.at[s1, :, pl.ds(ch * R, R)])
            return carry

        lax.fori_loop(0, per_w, task_body, 0)

    return gather_kernel


def kernel(token_ids, weight):
    S0, S1 = token_ids.shape
    V, D = weight.shape
    NW = 32
    R = 1024
    assert S0 % _L == 0 and (S1 * (S0 // R)) % NW == 0
    t2r = token_ids.T.reshape(S1, S0 // _L, _L)
    out_p = _make_gather_t(S1, S0, D, NW, R)(t2r, weight)
    return out_p.transpose(2, 0, 1)


# speedup vs baseline: 3.2498x; 1.0460x over previous
import functools

import jax
import jax.numpy as jnp
from jax import lax
from jax.experimental import pallas as pl
from jax.experimental.pallas import tpu as pltpu
from jax.experimental.pallas import tpu_sc as plsc

_L = 128


def _make_gather_t(S1, S0, D, NW, R):
    G = R // _L
    n_ch = S0 // R
    n_tasks = S1 * n_ch
    per_w = n_tasks // NW
    assert n_tasks % NW == 0
    mesh = plsc.VectorSubcoreMesh(core_axis_name="c", subcore_axis_name="s")

    assert per_w % 2 == 0 and per_w >= 4

    @functools.partial(
        pl.kernel,
        mesh=mesh,
        out_type=jax.ShapeDtypeStruct((S1, D, S0), jnp.float32),
        scratch_types=[
            pltpu.VMEM((2, G, _L), jnp.int32),
            pltpu.VMEM((2, R, D), jnp.float32),
            pltpu.VMEM((D, R), jnp.float32),
            pltpu.SemaphoreType.DMA,
            pltpu.SemaphoreType.DMA,
        ],
        compiler_params=pltpu.CompilerParams(
            use_tc_tiling_on_sc=False, needs_layout_passes=False),
    )
    def gather_kernel(t2_hbm, table_hbm, out_hbm, idx_v, rows_v, tr_v,
                      sg0, sg1):
        sg = (sg0, sg1)
        wid = lax.axis_index("s") * 2 + lax.axis_index("c")
        iota16 = lax.iota(jnp.int32, 16)

        def stage_and_fire(task, b):
            s1 = task // n_ch
            ch = task % n_ch
            pltpu.sync_copy(t2_hbm.at[s1, pl.ds(ch * G, G)], idx_v.at[b])
            for j in range(G):
                pltpu.make_async_copy(
                    table_hbm.at[idx_v.at[b, j]],
                    rows_v.at[b, pl.ds(j * _L, _L)],
                    sg[b],
                ).start()

        def retire(task, b):
            for j in range(G):
                pltpu.make_async_copy(
                    table_hbm.at[idx_v.at[b, j]],
                    rows_v.at[b, pl.ds(j * _L, _L)],
                    sg[b],
                ).wait()

            def c_body(c, _):
                col = jnp.full((16,), c, dtype=jnp.int32)

                def t_body(tb, __):
                    for u in range(4):
                        t0 = tb * 64 + u * 16
                        row = t0 + iota16
                        v = plsc.load_gather(rows_v.at[b], [row, col])
                        tr_v[c, pl.ds(t0, 16)] = v
                    return __

                lax.fori_loop(0, R // 64, t_body, 0)
                return _

            lax.fori_loop(0, D, c_body, 0)
            s1 = task // n_ch
            ch = task % n_ch
            pltpu.sync_copy(tr_v, out_hbm.at[s1, :, pl.ds(ch * R, R)])

        base = wid * per_w
        for b in (0, 1):
            stage_and_fire(base + b, b)

        def pair_body(p, carry):
            g0 = base + 2 * p
            for b in (0, 1):
                retire(g0 + b, b)
                stage_and_fire(g0 + b + 2, b)
            return carry

        lax.fori_loop(0, per_w // 2 - 1, pair_body, 0)
        for b in (0, 1):
            retire(base + per_w - 2 + b, b)

    return gather_kernel


def kernel(token_ids, weight):
    S0, S1 = token_ids.shape
    V, D = weight.shape
    NW = 32
    R = 1024
    assert S0 % _L == 0 and (S1 * (S0 // R)) % NW == 0
    t2r = token_ids.T.reshape(S1, S0 // _L, _L)
    out_p = _make_gather_t(S1, S0, D, NW, R)(t2r, weight)
    return out_p.transpose(2, 0, 1)


# speedup vs baseline: 3.3045x; 1.0168x over previous
import functools

import jax
import jax.numpy as jnp
from jax import lax
from jax.experimental import pallas as pl
from jax.experimental.pallas import tpu as pltpu
from jax.experimental.pallas import tpu_sc as plsc

_L = 128


def _make_gather_t(S1, S0, D, NW, R):
    G = R // _L
    n_ch = S0 // R
    n_tasks = S1 * n_ch
    per_w = n_tasks // NW
    assert n_tasks % NW == 0
    mesh = plsc.VectorSubcoreMesh(core_axis_name="c", subcore_axis_name="s")

    assert per_w % 2 == 0 and per_w >= 4

    @functools.partial(
        pl.kernel,
        mesh=mesh,
        out_type=jax.ShapeDtypeStruct((S1, D, S0), jnp.float32),
        scratch_types=[
            pltpu.VMEM((2, G, _L), jnp.int32),
            pltpu.VMEM((2, R, D), jnp.float32),
            pltpu.VMEM((D, R), jnp.float32),
            pltpu.SemaphoreType.DMA,
            pltpu.SemaphoreType.DMA,
        ],
        compiler_params=pltpu.CompilerParams(
            use_tc_tiling_on_sc=False, needs_layout_passes=False),
    )
    def gather_kernel(t2_hbm, table_hbm, out_hbm, idx_v, rows_v, tr_v,
                      sg0, sg1):
        sg = (sg0, sg1)
        wid = lax.axis_index("s") * 2 + lax.axis_index("c")
        iota16 = lax.iota(jnp.int32, 16)
        cols = [jnp.full((16,), c, dtype=jnp.int32) for c in range(D)]

        def stage_and_fire(task, b):
            s1 = task // n_ch
            ch = task % n_ch
            pltpu.sync_copy(t2_hbm.at[s1, pl.ds(ch * G, G)], idx_v.at[b])
            for j in range(G):
                pltpu.make_async_copy(
                    table_hbm.at[idx_v.at[b, j]],
                    rows_v.at[b, pl.ds(j * _L, _L)],
                    sg[b],
                ).start()

        def retire(task, b):
            for j in range(G):
                pltpu.make_async_copy(
                    table_hbm.at[idx_v.at[b, j]],
                    rows_v.at[b, pl.ds(j * _L, _L)],
                    sg[b],
                ).wait()

            def t_body(tb, __):
                t0 = tb * 16
                row = t0 + iota16
                for c in range(D):
                    v = plsc.load_gather(rows_v.at[b], [row, cols[c]])
                    tr_v[c, pl.ds(t0, 16)] = v
                return __

            lax.fori_loop(0, R // 16, t_body, 0)
            s1 = task // n_ch
            ch = task % n_ch
            pltpu.sync_copy(tr_v, out_hbm.at[s1, :, pl.ds(ch * R, R)])

        base = wid * per_w
        for b in (0, 1):
            stage_and_fire(base + b, b)

        def pair_body(p, carry):
            g0 = base + 2 * p
            for b in (0, 1):
                retire(g0 + b, b)
                stage_and_fire(g0 + b + 2, b)
            return carry

        lax.fori_loop(0, per_w // 2 - 1, pair_body, 0)
        for b in (0, 1):
            retire(base + per_w - 2 + b, b)

    return gather_kernel


def kernel(token_ids, weight):
    S0, S1 = token_ids.shape
    V, D = weight.shape
    NW = 32
    R = 1024
    assert S0 % _L == 0 and (S1 * (S0 // R)) % NW == 0
    t2r = token_ids.T.reshape(S1, S0 // _L, _L)
    out_p = _make_gather_t(S1, S0, D, NW, R)(t2r, weight)
    return out_p.transpose(2, 0, 1)


# speedup vs baseline: 5.5196x; 1.6703x over previous
import functools

import jax
import jax.numpy as jnp
from jax import lax
from jax.experimental import pallas as pl
from jax.experimental.pallas import tpu as pltpu
from jax.experimental.pallas import tpu_sc as plsc

_L = 128


def _make_gather_t(S1, S0, D, NW, R):
    G = R // _L
    n_ch = S0 // R
    n_tasks = S1 * n_ch
    per_w = n_tasks // NW
    assert n_tasks % NW == 0
    mesh = plsc.VectorSubcoreMesh(core_axis_name="c", subcore_axis_name="s")

    assert per_w % 2 == 0 and per_w >= 4

    @functools.partial(
        pl.kernel,
        mesh=mesh,
        out_type=jax.ShapeDtypeStruct((S1, D, S0), jnp.float32),
        scratch_types=[
            pltpu.VMEM((2, G, _L), jnp.int32),
            pltpu.VMEM((2, R, D), jnp.float32),
            pltpu.VMEM((D, R), jnp.float32),
            pltpu.SemaphoreType.DMA,
            pltpu.SemaphoreType.DMA,
        ],
        compiler_params=pltpu.CompilerParams(
            use_tc_tiling_on_sc=False, needs_layout_passes=False),
    )
    def gather_kernel(t2_hbm, table_hbm, out_hbm, idx_v, rows_v, tr_v,
                      sg0, sg1):
        sg = (sg0, sg1)
        wid = lax.axis_index("s") * 2 + lax.axis_index("c")
        iota16 = lax.iota(jnp.int32, 16)
        diag = [(c + lax.iota(jnp.int32, 16)) % D for c in range(D)]

        def stage_and_fire(task, b):
            s1 = task // n_ch
            ch = task % n_ch
            pltpu.sync_copy(t2_hbm.at[s1, pl.ds(ch * G, G)], idx_v.at[b])
            for j in range(G):
                pltpu.make_async_copy(
                    table_hbm.at[idx_v.at[b, j]],
                    rows_v.at[b, pl.ds(j * _L, _L)],
                    sg[b],
                ).start()

        def retire(task, b):
            for j in range(G):
                pltpu.make_async_copy(
                    table_hbm.at[idx_v.at[b, j]],
                    rows_v.at[b, pl.ds(j * _L, _L)],
                    sg[b],
                ).wait()

            def t_body(tb, __):
                row = tb * 16 + iota16
                for c in range(D):
                    v = plsc.load_gather(rows_v.at[b], [row, diag[c]])
                    plsc.store_scatter(tr_v, [diag[c], row], v)
                return __

            lax.fori_loop(0, R // 16, t_body, 0)
            s1 = task // n_ch
            ch = task % n_ch
            pltpu.sync_copy(tr_v, out_hbm.at[s1, :, pl.ds(ch * R, R)])

        base = wid * per_w
        for b in (0, 1):
            stage_and_fire(base + b, b)

        def pair_body(p, carry):
            g0 = base + 2 * p
            for b in (0, 1):
                retire(g0 + b, b)
                stage_and_fire(g0 + b + 2, b)
            return carry

        lax.fori_loop(0, per_w // 2 - 1, pair_body, 0)
        for b in (0, 1):
            retire(base + per_w - 2 + b, b)

    return gather_kernel


def kernel(token_ids, weight):
    S0, S1 = token_ids.shape
    V, D = weight.shape
    NW = 32
    R = 1024
    assert S0 % _L == 0 and (S1 * (S0 // R)) % NW == 0
    t2r = token_ids.T.reshape(S1, S0 // _L, _L)
    out_p = _make_gather_t(S1, S0, D, NW, R)(t2r, weight)
    return out_p.transpose(2, 0, 1)


# speedup vs baseline: 5.6689x; 1.0270x over previous
import functools

import jax
import jax.numpy as jnp
from jax import lax
from jax.experimental import pallas as pl
from jax.experimental.pallas import tpu as pltpu
from jax.experimental.pallas import tpu_sc as plsc

_L = 128
_NB = 4


def _make_gather_t(S1, S0, D, NW, R):
    G = R // _L
    n_ch = S0 // R
    n_tasks = S1 * n_ch
    per_w = n_tasks // NW
    assert n_tasks % NW == 0
    assert per_w % _NB == 0 and per_w >= 2 * _NB
    mesh = plsc.VectorSubcoreMesh(core_axis_name="c", subcore_axis_name="s")

    @functools.partial(
        pl.kernel,
        mesh=mesh,
        out_type=jax.ShapeDtypeStruct((S1, D, S0), jnp.float32),
        scratch_types=[
            pltpu.VMEM((_NB, G, _L), jnp.int32),
            pltpu.VMEM((_NB, R, D), jnp.float32),
            pltpu.VMEM((2, D, R), jnp.float32),
            [pltpu.SemaphoreType.DMA] * _NB,
            [pltpu.SemaphoreType.DMA] * 2,
        ],
        compiler_params=pltpu.CompilerParams(
            use_tc_tiling_on_sc=False, needs_layout_passes=False),
    )
    def gather_kernel(t2_hbm, table_hbm, out_hbm, idx_v, rows_v, tr_v,
                      sg, so):
        wid = lax.axis_index("s") * 2 + lax.axis_index("c")
        iota16 = lax.iota(jnp.int32, 16)
        diag = [(c + lax.iota(jnp.int32, 16)) % D for c in range(D)]

        def stage_and_fire(task, b):
            s1 = task // n_ch
            ch = task % n_ch
            pltpu.sync_copy(t2_hbm.at[s1, pl.ds(ch * G, G)], idx_v.at[b])
            for j in range(G):
                pltpu.make_async_copy(
                    table_hbm.at[idx_v.at[b, j]],
                    rows_v.at[b, pl.ds(j * _L, _L)],
                    sg[b],
                ).start()

        def wait_gathers(b):
            for j in range(G):
                pltpu.make_async_copy(
                    table_hbm.at[idx_v.at[b, j]],
                    rows_v.at[b, pl.ds(j * _L, _L)],
                    sg[b],
                ).wait()

        def transpose(b, tb):
            def t_body(t, __):
                row = t * 16 + iota16
                for c in range(D):
                    v = plsc.load_gather(rows_v.at[b], [row, diag[c]])
                    plsc.store_scatter(tr_v.at[tb], [diag[c], row], v)
                return __

            lax.fori_loop(0, R // 16, t_body, 0)

        def out_copy(task, tb):
            s1 = task // n_ch
            ch = task % n_ch
            return pltpu.make_async_copy(
                tr_v.at[tb], out_hbm.at[s1, :, pl.ds(ch * R, R)], so[tb])

        base = wid * per_w
        for b in range(_NB):
            stage_and_fire(base + b, b)

        for b in range(_NB):
            g = base + b
            tb = b % 2
            wait_gathers(b)
            if b >= 2:
                out_copy(g - 2, tb).wait()
            transpose(b, tb)
            out_copy(g, tb).start()
            stage_and_fire(g + _NB, b)

        def group_body(p, carry):
            g0 = base + _NB * p
            for b in range(_NB):
                g = g0 + b
                tb = b % 2
                wait_gathers(b)
                out_copy(g - 2, tb).wait()
                transpose(b, tb)
                out_copy(g, tb).start()
                stage_and_fire(g + _NB, b)
            return carry

        lax.fori_loop(1, per_w // _NB - 1, group_body, 0)

        for b in range(_NB):
            g = base + per_w - _NB + b
            tb = b % 2
            wait_gathers(b)
            out_copy(g - 2, tb).wait()
            transpose(b, tb)
            out_copy(g, tb).start()
        for b in (0, 1):
            out_copy(base + per_w - 2 + b, b).wait()

    return gather_kernel


def kernel(token_ids, weight):
    S0, S1 = token_ids.shape
    V, D = weight.shape
    NW = 32
    R = 512
    assert S0 % _L == 0 and (S1 * (S0 // R)) % NW == 0
    t2r = token_ids.T.reshape(S1, S0 // _L, _L)
    out_p = _make_gather_t(S1, S0, D, NW, R)(t2r, weight)
    return out_p.transpose(2, 0, 1)


# speedup vs baseline: 7.0920x; 1.2510x over previous
import functools

import jax
import jax.numpy as jnp
from jax import lax
from jax.experimental import pallas as pl
from jax.experimental.pallas import tpu as pltpu
from jax.experimental.pallas import tpu_sc as plsc

_L = 128
_NB = 4


def _make_gather_t(S1, S0, D, NW, R):
    G = R // _L
    n_ch = S0 // R
    n_tasks = S1 * n_ch
    per_w = n_tasks // NW
    assert n_tasks % NW == 0
    assert per_w % _NB == 0 and per_w >= 2 * _NB
    mesh = plsc.VectorSubcoreMesh(core_axis_name="c", subcore_axis_name="s")

    @functools.partial(
        pl.kernel,
        mesh=mesh,
        out_type=jax.ShapeDtypeStruct((S1, D, S0), jnp.float32),
        scratch_types=[
            pltpu.VMEM((_NB, G, _L), jnp.int32),
            pltpu.VMEM((_NB, R, D), jnp.float32),
            pltpu.VMEM((2, D, R), jnp.float32),
            [pltpu.SemaphoreType.DMA] * _NB,
            [pltpu.SemaphoreType.DMA] * 2,
        ],
        compiler_params=pltpu.CompilerParams(
            use_tc_tiling_on_sc=False, needs_layout_passes=False),
    )
    def gather_kernel(t2_hbm, table_hbm, out_hbm, idx_v, rows_v, tr_v,
                      sg, so):
        wid = lax.axis_index("s") * 2 + lax.axis_index("c")
        iota16 = lax.iota(jnp.int32, 16)
        diag = [(c + lax.iota(jnp.int32, 16)) % D for c in range(D)]

        def stage_and_fire(task, b):
            s1 = task // n_ch
            ch = task % n_ch
            pltpu.sync_copy(t2_hbm.at[s1, pl.ds(ch * G, G)], idx_v.at[b])
            for j in range(G):
                pltpu.make_async_copy(
                    table_hbm.at[idx_v.at[b, j]],
                    rows_v.at[b, pl.ds(j * _L, _L)],
                    sg[b],
                ).start()

        def wait_gathers(b):
            for j in range(G):
                pltpu.make_async_copy(
                    table_hbm.at[idx_v.at[b, j]],
                    rows_v.at[b, pl.ds(j * _L, _L)],
                    sg[b],
                ).wait()

        def transpose(b, tb):
            def t_body(t, __):
                row = t * 16 + iota16
                for c0 in range(0, D, 8):
                    vs = [plsc.load_gather(rows_v.at[b], [row, diag[c]])
                          for c in range(c0, c0 + 8)]
                    for k, c in enumerate(range(c0, c0 + 8)):
                        plsc.store_scatter(tr_v.at[tb], [diag[c], row], vs[k])
                return __

            lax.fori_loop(0, R // 16, t_body, 0)

        def out_copy(task, tb):
            s1 = task // n_ch
            ch = task % n_ch
            return pltpu.make_async_copy(
                tr_v.at[tb], out_hbm.at[s1, :, pl.ds(ch * R, R)], so[tb])

        def out_start(task, tb):
            out_copy(task, tb).start()

        def out_wait(task, tb):
            out_copy(task, tb).wait()

        base = wid * per_w
        for b in range(_NB):
            stage_and_fire(base + b, b)

        for b in range(_NB):
            g = base + b
            tb = b % 2
            wait_gathers(b)
            if b >= 2:
                out_wait(g - 2, tb)
            transpose(b, tb)
            out_start(g, tb)
            stage_and_fire(g + _NB, b)

        def group_body(p, carry):
            g0 = base + _NB * p
            for b in range(_NB):
                g = g0 + b
                tb = b % 2
                wait_gathers(b)
                out_wait(g - 2, tb)
                transpose(b, tb)
                out_start(g, tb)
                stage_and_fire(g + _NB, b)
            return carry

        lax.fori_loop(1, per_w // _NB - 1, group_body, 0)

        for b in range(_NB):
            g = base + per_w - _NB + b
            tb = b % 2
            wait_gathers(b)
            out_wait(g - 2, tb)
            transpose(b, tb)
            out_start(g, tb)
        for b in (0, 1):
            out_wait(base + per_w - 2 + b, b)

    return gather_kernel


def kernel(token_ids, weight):
    S0, S1 = token_ids.shape
    V, D = weight.shape
    NW = 32
    R = 512
    assert S0 % _L == 0 and (S1 * (S0 // R)) % NW == 0
    t2r = token_ids.T.reshape(S1, S0 // _L, _L)
    out_p = _make_gather_t(S1, S0, D, NW, R)(t2r, weight)
    return out_p.transpose(2, 0, 1)
